# baseline (speedup 1.0000x reference)
import functools
import jax
import jax.numpy as jnp
from jax.experimental import pallas as pl
from jax.experimental.pallas import tpu as pltpu

PAE_IN = 8
_EB = 512


def _pae_body(x_ref, w1_ref, b1_ref, g_ref, beta_ref, w2_ref, b2_ref, out_ref):
    x = x_ref[...]
    gscale = g_ref[...] * (1.0 / jnp.sqrt(1.0 + 1e-5))

    def parser(h):
        h = jnp.maximum(h @ w1_ref[...] + b1_ref[...], 0.0)
        h = h * gscale + beta_ref[...]
        return h @ w2_ref[...] + b2_ref[...]

    h1 = parser(x[:, :PAE_IN])
    h2 = parser(x[:, PAE_IN:])
    num = jnp.sum(h1 * h2, axis=1)
    n1 = jnp.sum(h1 * h1, axis=1)
    n2 = jnp.sum(h2 * h2, axis=1)
    den = jnp.maximum(jnp.sqrt(n1 * n2), 1e-8)
    out_ref[:, 0] = (num / den + 1.0) * 0.5


def _pae(x, w1, b1, g, beta, w2, b2):
    e = x.shape[0]
    grid = e // _EB
    out = pl.pallas_call(
        _pae_body,
        grid=(grid,),
        in_specs=[
            pl.BlockSpec((_EB, 2 * PAE_IN), lambda i: (i, 0)),
            pl.BlockSpec((PAE_IN, 128), lambda i: (0, 0)),
            pl.BlockSpec((128,), lambda i: (0,)),
            pl.BlockSpec((128,), lambda i: (0,)),
            pl.BlockSpec((128,), lambda i: (0,)),
            pl.BlockSpec((128, 128), lambda i: (0, 0)),
            pl.BlockSpec((128,), lambda i: (0,)),
        ],
        out_specs=pl.BlockSpec((_EB, 1), lambda i: (i, 0)),
        out_shape=jax.ShapeDtypeStruct((e, 1), jnp.float32),
    )(x, w1, b1, g, beta, w2, b2)
    return out.reshape(e)


def kernel(features, edge_index, edgenet_input,
           pae_w1, pae_b1, pae_g, pae_beta, pae_w2, pae_b2,
           cheb0_0, cheb0_1, cheb0_2,
           cheb1_0, cheb1_1, cheb1_2,
           cheb2_0, cheb2_1, cheb2_2,
           cheb3_0, cheb3_1, cheb3_2,
           cls_w1, cls_b1, cls_g, cls_beta, cls_w2, cls_b2):
    n = features.shape[0]
    ei = edge_index.astype(jnp.int32)
    src, dst = ei[0], ei[1]

    ew = _pae(edgenet_input, pae_w1, pae_b1, pae_g, pae_beta, pae_w2, pae_b2)

    deg = jax.ops.segment_sum(ew, src, num_segments=n)
    safe = jnp.where(deg > 0, deg, 1.0)
    dis = jnp.where(deg > 0, jax.lax.rsqrt(safe), 0.0)
    norm_w = -dis[src] * ew * dis[dst]

    def prop(h):
        return jax.ops.segment_sum(norm_w[:, None] * h[src], dst, num_segments=n)

    def cheb(x, w0, w1, w2):
        tx1 = prop(x)
        tx2 = 2.0 * prop(tx1) - x
        return x @ w0 + tx1 @ w1 + tx2 @ w2

    h = jnp.maximum(cheb(features, cheb0_0, cheb0_1, cheb0_2), 0.0)
    h0 = h
    for ws in ((cheb1_0, cheb1_1, cheb1_2), (cheb2_0, cheb2_1, cheb2_2),
               (cheb3_0, cheb3_1, cheb3_2)):
        h = jnp.maximum(cheb(h, *ws), 0.0)
        h0 = jnp.concatenate([h0, h], axis=1)
    jk = h0
    z = jnp.maximum(jk @ cls_w1 + cls_b1, 0.0)
    z = z * (1.0 / jnp.sqrt(1.0 + 1e-5)) * cls_g + cls_beta
    z = z @ cls_w2 + cls_b2
    logit = jax.nn.log_softmax(z, axis=1)
    return jk, logit


# speedup vs baseline: 4.0051x; 4.0051x over previous
import functools
import jax
import jax.numpy as jnp
from jax import lax
from jax.experimental import pallas as pl
from jax.experimental.pallas import tpu as pltpu
from jax.experimental.pallas import tpu_sc as plsc

PAE_IN = 8
_EB = 512
_NSUB = 32
_C = 128


def _pae_body(x_ref, w1_ref, b1_ref, g_ref, beta_ref, w2_ref, b2_ref, out_ref):
    x = x_ref[...]
    gscale = g_ref[...] * (1.0 / jnp.sqrt(1.0 + 1e-5))

    def parser(h):
        h = jnp.maximum(h @ w1_ref[...] + b1_ref[...], 0.0)
        h = h * gscale + beta_ref[...]
        return h @ w2_ref[...] + b2_ref[...]

    h1 = parser(x[:, :PAE_IN])
    h2 = parser(x[:, PAE_IN:])
    num = jnp.sum(h1 * h2, axis=1)
    n1 = jnp.sum(h1 * h1, axis=1)
    n2 = jnp.sum(h2 * h2, axis=1)
    den = jnp.maximum(jnp.sqrt(n1 * n2), 1e-8)
    out_ref[:, 0] = (num / den + 1.0) * 0.5


def _pae(x, w1, b1, g, beta, w2, b2):
    e = x.shape[0]
    grid = e // _EB
    out = pl.pallas_call(
        _pae_body,
        grid=(grid,),
        in_specs=[
            pl.BlockSpec((_EB, 2 * PAE_IN), lambda i: (i, 0)),
            pl.BlockSpec((PAE_IN, 128), lambda i: (0, 0)),
            pl.BlockSpec((128,), lambda i: (0,)),
            pl.BlockSpec((128,), lambda i: (0,)),
            pl.BlockSpec((128,), lambda i: (0,)),
            pl.BlockSpec((128, 128), lambda i: (0, 0)),
            pl.BlockSpec((128,), lambda i: (0,)),
        ],
        out_specs=pl.BlockSpec((_EB, 1), lambda i: (i, 0)),
        out_shape=jax.ShapeDtypeStruct((e, 1), jnp.float32),
    )(x, w1, b1, g, beta, w2, b2)
    return out.reshape(e)


@functools.lru_cache(maxsize=None)
def _make_prop(n_rows, np_rows, dh, ch):
    mesh = plsc.VectorSubcoreMesh(core_axis_name="c", subcore_axis_name="s")
    rows_per = np_rows // 16
    nblk = rows_per // _C

    @functools.partial(
        pl.kernel,
        out_type=jax.ShapeDtypeStruct((2, np_rows, dh), jnp.float32),
        mesh=mesh,
        compiler_params=pltpu.CompilerParams(use_tc_tiling_on_sc=False),
        scratch_types=[
            pltpu.VMEM((ch, _C), jnp.int32),
            pltpu.VMEM((ch, _C), jnp.int32),
            pltpu.VMEM((ch, _C), jnp.float32),
            pltpu.VMEM((_C, dh), jnp.float32),
            pltpu.VMEM_SHARED((np_rows, dh), jnp.float32),
            pltpu.SemaphoreType.DMA,
        ],
    )
    def prop_kernel(h_hbm, src_hbm, dst_hbm, w_hbm, out_hbm,
                    src_v, dst_v, w_v, buf, acc, sem):
        c = lax.axis_index("c")
        s = lax.axis_index("s")
        wid = s * 2 + c
        pltpu.sync_copy(src_hbm.at[wid], src_v)
        pltpu.sync_copy(dst_hbm.at[wid], dst_v)
        pltpu.sync_copy(w_hbm.at[wid], w_v)

        zero16 = jnp.zeros((16,), jnp.float32)

        def zrow(i, carry):
            for k in range(dh // 16):
                buf[i, pl.ds(k * 16, 16)] = zero16
            return carry

        lax.fori_loop(0, _C, zrow, 0)
        for b in range(nblk):
            pltpu.sync_copy(buf, acc.at[pl.ds(s * rows_per + b * _C, _C)])
        plsc.subcore_barrier()

        def chunk(j, carry):
            pltpu.async_copy(h_hbm.at[src_v.at[j]], buf, sem).wait()

            def group(g, gcarry):
                w16 = w_v[j, pl.ds(g * 16, 16)]
                for k in range(16):
                    ws = lax.broadcast(w16[k], (16,))
                    e = g * 16 + k
                    for q in range(dh // 16):
                        buf[e, pl.ds(q * 16, 16)] = buf[e, pl.ds(q * 16, 16)] * ws
                return gcarry

            lax.fori_loop(0, _C // 16, group, 0)
            pltpu.sync_copy(buf, acc.at[dst_v.at[j]], add=True)
            return carry

        lax.fori_loop(0, ch, chunk, 0)
        plsc.subcore_barrier()
        for b in range(nblk):
            pltpu.sync_copy(acc.at[pl.ds(s * rows_per + b * _C, _C)],
                            out_hbm.at[c, pl.ds(s * rows_per + b * _C, _C)])

    return prop_kernel


def kernel(features, edge_index, edgenet_input,
           pae_w1, pae_b1, pae_g, pae_beta, pae_w2, pae_b2,
           cheb0_0, cheb0_1, cheb0_2,
           cheb1_0, cheb1_1, cheb1_2,
           cheb2_0, cheb2_1, cheb2_2,
           cheb3_0, cheb3_1, cheb3_2,
           cls_w1, cls_b1, cls_g, cls_beta, cls_w2, cls_b2):
    n = features.shape[0]
    e = edge_index.shape[1]
    np_rows = ((n + 2047) // 2048) * 2048
    ch = -(-e // (_NSUB * _C))
    ep = _NSUB * ch * _C

    ei = edge_index.astype(jnp.int32)
    pad = ep - e
    src3 = jnp.pad(ei[0], (0, pad)).reshape(_NSUB, ch, _C)
    dst3 = jnp.pad(ei[1], (0, pad)).reshape(_NSUB, ch, _C)

    ew = _pae(edgenet_input, pae_w1, pae_b1, pae_g, pae_beta, pae_w2, pae_b2)
    ew3 = jnp.pad(ew, (0, pad)).reshape(_NSUB, ch, _C)

    prop_k = lambda dh: _make_prop(n, np_rows, dh, ch)

    ones16 = jnp.ones((n, 16), jnp.float32)
    degp = prop_k(16)(ones16, src3, src3, ew3)
    deg = degp[0, :n, 0] + degp[1, :n, 0]
    dis = jnp.where(deg > 0, lax.rsqrt(jnp.where(deg > 0, deg, 1.0)), 0.0)
    disc = dis[:, None]

    def prop(h):
        p = prop_k(h.shape[1])(-(disc * h), src3, dst3, ew3)
        return disc * (p[0] + p[1])[:n]

    def cheb(x, w0, w1, w2):
        tx1 = prop(x)
        tx2 = 2.0 * prop(tx1) - x
        return x @ w0 + tx1 @ w1 + tx2 @ w2

    h = jnp.maximum(cheb(features, cheb0_0, cheb0_1, cheb0_2), 0.0)
    h0 = h
    for ws in ((cheb1_0, cheb1_1, cheb1_2), (cheb2_0, cheb2_1, cheb2_2),
               (cheb3_0, cheb3_1, cheb3_2)):
        h = jnp.maximum(cheb(h, *ws), 0.0)
        h0 = jnp.concatenate([h0, h], axis=1)
    jk = h0
    z = jnp.maximum(jk @ cls_w1 + cls_b1, 0.0)
    z = z * (1.0 / jnp.sqrt(1.0 + 1e-5)) * cls_g + cls_beta
    z = z @ cls_w2 + cls_b2
    logit = jax.nn.log_softmax(z, axis=1)
    return jk, logit
